# Initial kernel scaffold; baseline (speedup 1.0000x reference)
import jax
import jax.numpy as jnp
from jax.experimental import pallas as pl


def kernel(pts, mlp1_w0, mlp1_b0, mlp1_rw1, mlp1_rb1, mlp1_rw2, mlp1_rb2, lin1_rw, lin1_rb, lin1_w, lin1_b, lin2_rw, lin2_rb, lin2_w, lin2_b, mlp2_w0, mlp2_b0, mlp2_w1, mlp2_b1):
    raise NotImplementedError("write your pallas kernel here")


# speedup vs baseline: 15.9417x; 15.9417x over previous
import functools

import jax
import jax.numpy as jnp
from jax import lax
from jax.experimental import pallas as pl
from jax.experimental.pallas import tpu as pltpu
from jax.experimental.pallas import tpu_sc as plsc

_B, _N, _K = 16, 2048, 16
_RK = 256
_RM = 2048
_RF = 512


def _knn_cov_body(xrow_ref, xful_ref, xt_ref, idx_ref, cov_ref):
    b = pl.program_id(0)
    xb = xrow_ref[0]
    xf = xful_ref[0]
    xt = xt_ref[0]
    inner = -2.0 * jnp.dot(xb, xt, preferred_element_type=jnp.float32)
    xx_rows = jnp.sum(xb * xb, axis=1, keepdims=True)
    xx_cols = jnp.sum(xt * xt, axis=0, keepdims=True)
    pd = -xx_rows - inner - xx_cols
    iota = lax.broadcasted_iota(jnp.int32, (_RK, _N), 1)
    neg_inf = jnp.float32(float("-inf"))
    vals = pd
    cols = []
    sels = []
    for k in range(_K):
        m = jnp.max(vals, axis=1, keepdims=True)
        cand = jnp.where(vals == m, iota, _N)
        a = jnp.min(cand, axis=1, keepdims=True)
        cols.append(a)
        sel = iota == a
        if k < 2:
            sels.append(sel.astype(jnp.float32))
        if k < _K - 1:
            vals = jnp.where(sel, neg_inf, vals)
    idx_ref[0] = jnp.concatenate(cols, axis=1) + b * _N
    nbr0 = jnp.dot(sels[0], xf, preferred_element_type=jnp.float32)
    nbr1 = jnp.dot(sels[1], xf, preferred_element_type=jnp.float32)
    pieces = [xb]
    for i2 in range(3):
        for j2 in range(3):
            pieces.append(nbr0[:, i2:i2 + 1] * nbr1[:, j2:j2 + 1])
    cov_ref[0] = jnp.concatenate(pieces, axis=1)


def _knn_cov(pts):
    xt = jnp.transpose(pts, (0, 2, 1))
    return pl.pallas_call(
        _knn_cov_body,
        grid=(_B, _N // _RK),
        in_specs=[
            pl.BlockSpec((1, _RK, 3), lambda b, i: (b, i, 0)),
            pl.BlockSpec((1, _N, 3), lambda b, i: (b, 0, 0)),
            pl.BlockSpec((1, 3, _N), lambda b, i: (b, 0, 0)),
        ],
        out_specs=[
            pl.BlockSpec((1, _RK, _K), lambda b, i: (b, i, 0)),
            pl.BlockSpec((1, _RK, 12), lambda b, i: (b, i, 0)),
        ],
        out_shape=[
            jax.ShapeDtypeStruct((_B, _N, _K), jnp.int32),
            jax.ShapeDtypeStruct((_B, _N, 12), jnp.float32),
        ],
    )(pts, pts, xt)


def _mlp1_body(x_ref, w0_ref, b0_ref, rw1_ref, rb1_ref, rw2_ref, rb2_ref,
               o_ref):
    x = x_ref[...]
    h = jax.nn.relu(
        jnp.dot(x, w0_ref[...], preferred_element_type=jnp.float32)
        + b0_ref[...])
    h = jax.nn.relu(h + (
        jnp.dot(h, rw1_ref[...], preferred_element_type=jnp.float32)
        + rb1_ref[...]))
    h = jax.nn.relu(h + (
        jnp.dot(h, rw2_ref[...], preferred_element_type=jnp.float32)
        + rb2_ref[...]))
    o_ref[...] = h


def _mlp1(cov2, w0, b0, rw1, rb1, rw2, rb2):
    m = _B * _N
    full = lambda a: pl.BlockSpec(a.shape, lambda i: tuple(0 for _ in a.shape))
    return pl.pallas_call(
        _mlp1_body,
        grid=(m // _RM,),
        in_specs=[pl.BlockSpec((_RM, 12), lambda i: (i, 0)),
                  full(w0), full(b0), full(rw1), full(rb1),
                  full(rw2), full(rb2)],
        out_specs=pl.BlockSpec((_RM, 128), lambda i: (i, 0)),
        out_shape=jax.ShapeDtypeStruct((m, 128), jnp.float32),
    )(cov2, w0, b0, rw1, rb1, rw2, rb2)


def _pool_max(feat_flat, idx_flat, chans):
    m = feat_flat.shape[0]
    nw = 32
    pw = m // nw
    ch = 8
    mesh = plsc.VectorSubcoreMesh(core_axis_name="c", subcore_axis_name="s")

    @functools.partial(
        pl.kernel, mesh=mesh,
        out_type=jax.ShapeDtypeStruct((m, chans), jnp.float32),
        scratch_types=[
            pltpu.VMEM((ch * _K,), jnp.int32),
            pltpu.VMEM((ch * _K, chans), jnp.float32),
            pltpu.VMEM((ch, chans), jnp.float32),
            pltpu.SemaphoreType.DMA,
        ],
    )
    def sc_pool(feat_hbm, idx_hbm, out_hbm, idx_v, rows_v, out_v, sem):
        wid = lax.axis_index("s") * 2 + lax.axis_index("c")
        base = wid * pw

        @pl.loop(0, pw, step=ch)
        def _chunk(o0):
            start = base + o0
            pltpu.sync_copy(idx_hbm.at[pl.ds(start * _K, ch * _K)], idx_v)
            pltpu.async_copy(feat_hbm.at[idx_v], rows_v, sem).wait()

            @pl.loop(0, ch)
            def _row(o):
                r0 = o * _K
                for c in range(chans // 16):
                    sl = pl.ds(c * 16, 16)
                    acc = rows_v[r0, sl]
                    for r in range(1, _K):
                        acc = jnp.maximum(acc, rows_v[r0 + r, sl])
                    out_v[o, sl] = acc

            pltpu.sync_copy(out_v, out_hbm.at[pl.ds(start, ch)])

    return sc_pool(feat_flat, idx_flat)


def _graph1_body(x_ref, rw_ref, rb_ref, w_ref, b_ref, o_ref):
    x = x_ref[...]
    x = jax.nn.relu(x + (
        jnp.dot(x, rw_ref[...], preferred_element_type=jnp.float32)
        + rb_ref[...]))
    o_ref[...] = jax.nn.relu(
        jnp.dot(x, w_ref[...], preferred_element_type=jnp.float32)
        + b_ref[...])


def _graph1(pooled1, rw, rb, w, b):
    m = _B * _N
    full = lambda a: pl.BlockSpec(a.shape, lambda i: tuple(0 for _ in a.shape))
    return pl.pallas_call(
        _graph1_body,
        grid=(m // _RM,),
        in_specs=[pl.BlockSpec((_RM, 128), lambda i: (i, 0)),
                  full(rw), full(rb), full(w), full(b)],
        out_specs=pl.BlockSpec((_RM, 256), lambda i: (i, 0)),
        out_shape=jax.ShapeDtypeStruct((m, 256), jnp.float32),
    )(pooled1, rw, rb, w, b)


def _final_body(p_ref, rw_ref, rb_ref, w_ref, b_ref, w0_ref, b0_ref,
                w1_ref, b1_ref, o_ref, acc_ref):
    j = pl.program_id(1)
    x = p_ref[0]
    x = jax.nn.relu(x + (
        jnp.dot(x, rw_ref[...], preferred_element_type=jnp.float32)
        + rb_ref[...]))
    f3 = (jnp.dot(x, w_ref[...], preferred_element_type=jnp.float32)
          + b_ref[...])
    bm = jnp.max(f3, axis=0, keepdims=True)

    @pl.when(j == 0)
    def _():
        acc_ref[...] = bm

    @pl.when(j != 0)
    def _():
        acc_ref[...] = jnp.maximum(acc_ref[...], bm)

    @pl.when(j == (_N // _RF) - 1)
    def _():
        g = acc_ref[...]
        f = jax.nn.relu(
            jnp.dot(g, w0_ref[...], preferred_element_type=jnp.float32)
            + b0_ref[...])
        o_ref[0] = (jnp.dot(f, w1_ref[...],
                            preferred_element_type=jnp.float32)
                    + b1_ref[...])


def _final(pooled2, rw, rb, w, b, w0, b0, w1, b1):
    full = lambda a: pl.BlockSpec(
        a.shape, lambda bb, j: tuple(0 for _ in a.shape))
    return pl.pallas_call(
        _final_body,
        grid=(_B, _N // _RF),
        in_specs=[pl.BlockSpec((1, _RF, 256), lambda bb, j: (bb, j, 0)),
                  full(rw), full(rb), full(w), full(b),
                  full(w0), full(b0), full(w1), full(b1)],
        out_specs=pl.BlockSpec((1, 1, 1024), lambda bb, j: (bb, 0, 0)),
        out_shape=jax.ShapeDtypeStruct((_B, 1, 1024), jnp.float32),
        scratch_shapes=[pltpu.VMEM((1, 512), jnp.float32)],
    )(pooled2, rw, rb, w, b, w0, b0, w1, b1)


def kernel(pts, mlp1_w0, mlp1_b0, mlp1_rw1, mlp1_rb1, mlp1_rw2, mlp1_rb2,
           lin1_rw, lin1_rb, lin1_w, lin1_b, lin2_rw, lin2_rb, lin2_w,
           lin2_b, mlp2_w0, mlp2_b0, mlp2_w1, mlp2_b1):
    idx3, cov = _knn_cov(pts)
    idxf = idx3.reshape(_B * _N * _K)
    cov2 = cov.reshape(_B * _N, 12)
    feat1 = _mlp1(cov2, mlp1_w0, mlp1_b0.reshape(1, 128),
                  mlp1_rw1, mlp1_rb1.reshape(1, 128),
                  mlp1_rw2, mlp1_rb2.reshape(1, 128))
    pooled1 = _pool_max(feat1, idxf, 128)
    feat2 = _graph1(pooled1, lin1_rw, lin1_rb.reshape(1, 128),
                    lin1_w, lin1_b.reshape(1, 256))
    pooled2 = _pool_max(feat2, idxf, 256)
    feat = _final(pooled2.reshape(_B, _N, 256),
                  lin2_rw, lin2_rb.reshape(1, 256),
                  lin2_w, lin2_b.reshape(1, 512),
                  mlp2_w0, mlp2_b0.reshape(1, 1024),
                  mlp2_w1, mlp2_b1.reshape(1, 1024))
    return feat, feat1.reshape(_B, _N, 128)
